# baseline (speedup 1.0000x reference)
import jax
import jax.numpy as jnp
from jax import lax
from jax.experimental import pallas as pl
from jax.experimental.pallas import tpu as pltpu
from jax.experimental.pallas import tpu_sc as plsc

_B = 16
_C = 3
_H = 512
_W = 512

_OFF_WC1 = 0
_OFF_WI1 = 100
_OFF_WC2 = 10100
_OFF_WI2 = 10150
_OFF_BC1 = 15150
_OFF_BI1 = 15250
_OFF_BC2 = 25250
_OFF_BI2 = 25300
_ROWS = 30304

_P = 4
_NBUF = 4
_NCHUNK = (_B * _C) // _P


def _sc_lookup(cam_hbm, idi_hbm, dt_hbm, tbl_hbm,
               w_out, b_out,
               cam_v, idi_v, dt_v, ic_v, ii_v, g_cam, g_id, r_v, sem):
    wid = lax.axis_index("s") * 2 + lax.axis_index("c")

    def lookup(off_cam1, off_id1, off_cam2, off_id2, out_ref):
        cp = [pltpu.async_copy(src, dst, sem)
              for src, dst in ((cam_hbm, cam_v), (idi_hbm, idi_v),
                               (dt_hbm, dt_v))]
        for c in cp:
            c.wait()
        use1 = dt_v[...] == 0
        ic_v[...] = cam_v[...] + jnp.where(use1, off_cam1, off_cam2)
        ii_v[...] = idi_v[...] + jnp.where(use1, off_id1, off_id2)
        g1 = pltpu.async_copy(tbl_hbm.at[ic_v], g_cam, sem)
        g2 = pltpu.async_copy(tbl_hbm.at[ii_v], g_id, sem)
        g1.wait()
        g2.wait()

        samp = lax.iota(jnp.int32, _B)
        for c in range(_C):
            cvec = jnp.full((_B,), c, jnp.int32)
            v = (plsc.load_gather(g_cam, [samp, cvec])
                 + plsc.load_gather(g_id, [samp, cvec]))
            plsc.store_scatter(r_v, [samp * _C + c], v)
        pltpu.sync_copy(r_v, out_ref)

    @pl.when(wid == 0)
    def _():
        lookup(_OFF_WC1, _OFF_WI1, _OFF_WC2, _OFF_WI2, w_out)

    @pl.when(wid == 1)
    def _():
        lookup(_OFF_BC1, _OFF_BI1, _OFF_BC2, _OFF_BI2, b_out)


def _affine_body(w_ref, b_ref, img_ref, out_ref, buf_in, buf_out, sem_in, sem_out):
    def in_copy(k, s):
        return pltpu.make_async_copy(
            img_ref.at[pl.ds(k * _P, _P)], buf_in.at[s], sem_in.at[s])

    def out_copy(k, s):
        return pltpu.make_async_copy(
            buf_out.at[s], out_ref.at[pl.ds(k * _P, _P)], sem_out.at[s])

    for s in range(_NBUF):
        in_copy(s, s).start()
    for k in range(_NCHUNK):
        s = k % _NBUF
        in_copy(k, s).wait()
        if k >= _NBUF:
            out_copy(k - _NBUF, s).wait()
        w = w_ref[pl.ds(k * _P, _P)].reshape(_P, 1, 1)
        b = b_ref[pl.ds(k * _P, _P)].reshape(_P, 1, 1)
        buf_out[s] = buf_in[s] * w + b
        out_copy(k, s).start()
        if k + _NBUF < _NCHUNK:
            in_copy(k + _NBUF, s).start()
    for k in range(_NCHUNK - _NBUF, _NCHUNK):
        out_copy(k, k % _NBUF).wait()


def kernel(image, camindex, idindex, dataset_type,
           wcam1, bcam1, wident1, bident1,
           wcam2, bcam2, wident2, bident2):
    lanepad = ((0, 0), (0, 128 - _C))
    tbl = jnp.concatenate(
        [jnp.pad(t, lanepad) for t in (wcam1, wident1, wcam2, wident2,
                                       bcam1, bident1, bcam2, bident2)]
        + [jnp.zeros((4, 128), jnp.float32)], axis=0)

    mesh = plsc.VectorSubcoreMesh(core_axis_name="c", subcore_axis_name="s")
    vec = jax.ShapeDtypeStruct((_B * _C,), jnp.float32)
    sc_fn = pl.kernel(
        _sc_lookup,
        out_type=[vec, vec],
        mesh=mesh,
        scratch_types=[
            pltpu.VMEM((_B,), jnp.int32),
            pltpu.VMEM((_B,), jnp.int32),
            pltpu.VMEM((_B,), jnp.int32),
            pltpu.VMEM((_B,), jnp.int32),
            pltpu.VMEM((_B,), jnp.int32),
            pltpu.VMEM((_B, 128), jnp.float32),
            pltpu.VMEM((_B, 128), jnp.float32),
            pltpu.VMEM((_B * _C,), jnp.float32),
            pltpu.SemaphoreType.DMA,
        ],
        name="colorcal_sc_lookup",
        compiler_params=pltpu.CompilerParams(needs_layout_passes=False,
                                             use_tc_tiling_on_sc=True),
    )
    w48, b48 = sc_fn(camindex, idindex, dataset_type, tbl)

    img = image.reshape(_B * _C, _H, _W)
    out = pl.pallas_call(
        _affine_body,
        in_specs=[
            pl.BlockSpec(memory_space=pltpu.VMEM),
            pl.BlockSpec(memory_space=pltpu.VMEM),
            pl.BlockSpec(memory_space=pl.ANY),
        ],
        out_specs=pl.BlockSpec(memory_space=pl.ANY),
        out_shape=jax.ShapeDtypeStruct((_B * _C, _H, _W), jnp.float32),
        scratch_shapes=[
            pltpu.VMEM((_NBUF, _P, _H, _W), jnp.float32),
            pltpu.VMEM((_NBUF, _P, _H, _W), jnp.float32),
            pltpu.SemaphoreType.DMA((_NBUF,)),
            pltpu.SemaphoreType.DMA((_NBUF,)),
        ],
        name="colorcal_affine",
    )(w48, b48, img)
    return out.reshape(_B, _C, _H, _W)


# speedup vs baseline: 1.4440x; 1.4440x over previous
import jax
import jax.numpy as jnp
from jax import lax
from jax.experimental import pallas as pl
from jax.experimental.pallas import tpu as pltpu
from jax.experimental.pallas import tpu_sc as plsc

_B = 16
_C = 3
_H = 512
_W = 512

_OFF_WC1 = 0
_OFF_WI1 = 104
_OFF_WC2 = 10104
_OFF_WI2 = 10160
_OFF_BC1 = 15160
_OFF_BI1 = 15264
_OFF_BC2 = 25264
_OFF_BI2 = 25320
_ROWS = 30320

_P = 4
_NBUF = 4
_NCHUNK = (_B * _C) // _P


def _sc_lookup(cam_hbm, idi_hbm, dt_hbm, tbl_hbm,
               w_out, b_out,
               cam_v, idi_v, dt_v, ic_v, ii_v, g_cam, g_id, r_v, sem):
    wid = lax.axis_index("s") * 2 + lax.axis_index("c")

    def lookup(off_cam1, off_id1, off_cam2, off_id2, out_ref):
        cp = [pltpu.async_copy(src, dst, sem)
              for src, dst in ((cam_hbm, cam_v), (idi_hbm, idi_v),
                               (dt_hbm, dt_v))]
        for c in cp:
            c.wait()
        use1 = dt_v[...] == 0
        ic_v[...] = cam_v[...] + jnp.where(use1, off_cam1, off_cam2)
        ii_v[...] = idi_v[...] + jnp.where(use1, off_id1, off_id2)
        g1 = pltpu.async_copy(tbl_hbm.at[ic_v], g_cam, sem)
        g2 = pltpu.async_copy(tbl_hbm.at[ii_v], g_id, sem)
        g1.wait()
        g2.wait()

        samp = lax.iota(jnp.int32, _B)
        for c in range(_C):
            cvec = jnp.full((_B,), c, jnp.int32)
            v = (plsc.load_gather(g_cam, [samp, cvec])
                 + plsc.load_gather(g_id, [samp, cvec]))
            plsc.store_scatter(r_v, [samp * _C + c], v)
        pltpu.sync_copy(r_v, out_ref)

    @pl.when(wid == 0)
    def _():
        lookup(_OFF_WC1, _OFF_WI1, _OFF_WC2, _OFF_WI2, w_out)

    @pl.when(wid == 1)
    def _():
        lookup(_OFF_BC1, _OFF_BI1, _OFF_BC2, _OFF_BI2, b_out)


def _affine_body(w_ref, b_ref, img_ref, out_ref, buf_in, buf_out, sem_in, sem_out):
    def in_copy(k, s):
        return pltpu.make_async_copy(
            img_ref.at[pl.ds(k * _P, _P)], buf_in.at[s], sem_in.at[s])

    def out_copy(k, s):
        return pltpu.make_async_copy(
            buf_out.at[s], out_ref.at[pl.ds(k * _P, _P)], sem_out.at[s])

    for s in range(_NBUF):
        in_copy(s, s).start()
    for k in range(_NCHUNK):
        s = k % _NBUF
        in_copy(k, s).wait()
        if k >= _NBUF:
            out_copy(k - _NBUF, s).wait()
        w = w_ref[pl.ds(k * _P, _P)].reshape(_P, 1, 1)
        b = b_ref[pl.ds(k * _P, _P)].reshape(_P, 1, 1)
        buf_out[s] = buf_in[s] * w + b
        out_copy(k, s).start()
        if k + _NBUF < _NCHUNK:
            in_copy(k + _NBUF, s).start()
    for k in range(_NCHUNK - _NBUF, _NCHUNK):
        out_copy(k, k % _NBUF).wait()


def _stage_body(wc1, wi1, wc2, wi2, bc1, bi1, bc2, bi2, out_ref):
    for off, ref in ((_OFF_WC1, wc1), (_OFF_WI1, wi1),
                     (_OFF_WC2, wc2), (_OFF_WI2, wi2),
                     (_OFF_BC1, bc1), (_OFF_BI1, bi1),
                     (_OFF_BC2, bc2), (_OFF_BI2, bi2)):
        n = ref.shape[0]
        out_ref[pl.ds(off, n), pl.ds(0, _C)] = ref[...]


def kernel(image, camindex, idindex, dataset_type,
           wcam1, bcam1, wident1, bident1,
           wcam2, bcam2, wident2, bident2):
    tbl = pl.pallas_call(
        _stage_body,
        out_shape=jax.ShapeDtypeStruct((_ROWS, 128), jnp.float32),
        name="colorcal_stage_tables",
    )(wcam1, wident1, wcam2, wident2, bcam1, bident1, bcam2, bident2)

    mesh = plsc.VectorSubcoreMesh(core_axis_name="c", subcore_axis_name="s")
    vec = jax.ShapeDtypeStruct((_B * _C,), jnp.float32)
    sc_fn = pl.kernel(
        _sc_lookup,
        out_type=[vec, vec],
        mesh=mesh,
        scratch_types=[
            pltpu.VMEM((_B,), jnp.int32),
            pltpu.VMEM((_B,), jnp.int32),
            pltpu.VMEM((_B,), jnp.int32),
            pltpu.VMEM((_B,), jnp.int32),
            pltpu.VMEM((_B,), jnp.int32),
            pltpu.VMEM((_B, 128), jnp.float32),
            pltpu.VMEM((_B, 128), jnp.float32),
            pltpu.VMEM((_B * _C,), jnp.float32),
            pltpu.SemaphoreType.DMA,
        ],
        name="colorcal_sc_lookup",
        compiler_params=pltpu.CompilerParams(needs_layout_passes=False,
                                             use_tc_tiling_on_sc=True),
    )
    w48, b48 = sc_fn(camindex, idindex, dataset_type, tbl)

    img = image.reshape(_B * _C, _H, _W)
    out = pl.pallas_call(
        _affine_body,
        in_specs=[
            pl.BlockSpec(memory_space=pltpu.VMEM),
            pl.BlockSpec(memory_space=pltpu.VMEM),
            pl.BlockSpec(memory_space=pl.ANY),
        ],
        out_specs=pl.BlockSpec(memory_space=pl.ANY),
        out_shape=jax.ShapeDtypeStruct((_B * _C, _H, _W), jnp.float32),
        scratch_shapes=[
            pltpu.VMEM((_NBUF, _P, _H, _W), jnp.float32),
            pltpu.VMEM((_NBUF, _P, _H, _W), jnp.float32),
            pltpu.SemaphoreType.DMA((_NBUF,)),
            pltpu.SemaphoreType.DMA((_NBUF,)),
        ],
        name="colorcal_affine",
    )(w48, b48, img)
    return out.reshape(_B, _C, _H, _W)


# speedup vs baseline: 1.6771x; 1.1615x over previous
import jax
import jax.numpy as jnp
from jax import lax
from jax.experimental import pallas as pl
from jax.experimental.pallas import tpu as pltpu
from jax.experimental.pallas import tpu_sc as plsc

_B = 16
_C = 3
_H = 512
_W = 512

_P = 4
_NBUF = 4
_NCHUNK = (_B * _C) // _P


def _sc_lookup(cam_hbm, idi_hbm, dt_hbm,
               wc1_hbm, bc1_hbm, wi1_hbm, bi1_hbm,
               wc2_hbm, bc2_hbm, wi2_hbm, bi2_hbm,
               w_out, b_out,
               cam_v, idi_v, dt_v,
               g_cam, g_id, r_v, sem):
    wid = lax.axis_index("s") * 2 + lax.axis_index("c")

    def lookup(cam_tbl1, id_tbl1, cam_tbl2, id_tbl2, out_ref):
        cp = [pltpu.async_copy(src, dst, sem)
              for src, dst in ((cam_hbm, cam_v), (idi_hbm, idi_v),
                               (dt_hbm, dt_v))]
        for c in cp:
            c.wait()
        cam = cam_v[...]
        idi = idi_v[...]
        dt = dt_v[...]
        for i in range(_B):
            cam_i = cam[i]
            idi_i = idi[i]
            use1 = dt[i] == 0

            @pl.when(use1)
            def _():
                pltpu.async_copy(cam_tbl1.at[pl.ds(cam_i, 1)],
                                 g_cam.at[pl.ds(i, 1)], sem)
                pltpu.async_copy(id_tbl1.at[pl.ds(idi_i, 1)],
                                 g_id.at[pl.ds(i, 1)], sem)

            @pl.when(jnp.logical_not(use1))
            def _():
                pltpu.async_copy(cam_tbl2.at[pl.ds(cam_i, 1)],
                                 g_cam.at[pl.ds(i, 1)], sem)
                pltpu.async_copy(id_tbl2.at[pl.ds(idi_i, 1)],
                                 g_id.at[pl.ds(i, 1)], sem)
        for i in range(_B):
            pltpu.make_async_copy(cam_tbl1.at[pl.ds(0, 1)],
                                  g_cam.at[pl.ds(i, 1)], sem).wait()
            pltpu.make_async_copy(id_tbl1.at[pl.ds(0, 1)],
                                  g_id.at[pl.ds(i, 1)], sem).wait()

        samp = lax.iota(jnp.int32, _B)
        for c in range(_C):
            cvec = jnp.full((_B,), c, jnp.int32)
            v = (plsc.load_gather(g_cam, [samp, cvec])
                 + plsc.load_gather(g_id, [samp, cvec]))
            plsc.store_scatter(r_v, [samp * _C + c], v)
        pltpu.sync_copy(r_v, out_ref)

    @pl.when(wid == 0)
    def _():
        lookup(wc1_hbm, wi1_hbm, wc2_hbm, wi2_hbm, w_out)

    @pl.when(wid == 1)
    def _():
        lookup(bc1_hbm, bi1_hbm, bc2_hbm, bi2_hbm, b_out)


def _affine_body(w_ref, b_ref, img_ref, out_ref, buf_in, buf_out, sem_in, sem_out):
    def in_copy(k, s):
        return pltpu.make_async_copy(
            img_ref.at[pl.ds(k * _P, _P)], buf_in.at[s], sem_in.at[s])

    def out_copy(k, s):
        return pltpu.make_async_copy(
            buf_out.at[s], out_ref.at[pl.ds(k * _P, _P)], sem_out.at[s])

    for s in range(_NBUF):
        in_copy(s, s).start()
    for k in range(_NCHUNK):
        s = k % _NBUF
        in_copy(k, s).wait()
        if k >= _NBUF:
            out_copy(k - _NBUF, s).wait()
        w = w_ref[pl.ds(k * _P, _P)].reshape(_P, 1, 1)
        b = b_ref[pl.ds(k * _P, _P)].reshape(_P, 1, 1)
        buf_out[s] = buf_in[s] * w + b
        out_copy(k, s).start()
        if k + _NBUF < _NCHUNK:
            in_copy(k + _NBUF, s).start()
    for k in range(_NCHUNK - _NBUF, _NCHUNK):
        out_copy(k, k % _NBUF).wait()


def kernel(image, camindex, idindex, dataset_type,
           wcam1, bcam1, wident1, bident1,
           wcam2, bcam2, wident2, bident2):
    mesh = plsc.VectorSubcoreMesh(core_axis_name="c", subcore_axis_name="s")
    vec = jax.ShapeDtypeStruct((_B * _C,), jnp.float32)
    sc_fn = pl.kernel(
        _sc_lookup,
        out_type=[vec, vec],
        mesh=mesh,
        scratch_types=[
            pltpu.VMEM((_B,), jnp.int32),
            pltpu.VMEM((_B,), jnp.int32),
            pltpu.VMEM((_B,), jnp.int32),
            pltpu.VMEM((_B, _C), jnp.float32),
            pltpu.VMEM((_B, _C), jnp.float32),
            pltpu.VMEM((_B * _C,), jnp.float32),
            pltpu.SemaphoreType.DMA,
        ],
        name="colorcal_sc_lookup",
        compiler_params=pltpu.CompilerParams(needs_layout_passes=False,
                                             use_tc_tiling_on_sc=True),
    )
    w48, b48 = sc_fn(camindex, idindex, dataset_type,
                     wcam1, bcam1, wident1, bident1,
                     wcam2, bcam2, wident2, bident2)

    img = image.reshape(_B * _C, _H, _W)
    out = pl.pallas_call(
        _affine_body,
        in_specs=[
            pl.BlockSpec(memory_space=pltpu.VMEM),
            pl.BlockSpec(memory_space=pltpu.VMEM),
            pl.BlockSpec(memory_space=pl.ANY),
        ],
        out_specs=pl.BlockSpec(memory_space=pl.ANY),
        out_shape=jax.ShapeDtypeStruct((_B * _C, _H, _W), jnp.float32),
        scratch_shapes=[
            pltpu.VMEM((_NBUF, _P, _H, _W), jnp.float32),
            pltpu.VMEM((_NBUF, _P, _H, _W), jnp.float32),
            pltpu.SemaphoreType.DMA((_NBUF,)),
            pltpu.SemaphoreType.DMA((_NBUF,)),
        ],
        name="colorcal_affine",
    )(w48, b48, img)
    return out.reshape(_B, _C, _H, _W)


# speedup vs baseline: 1.6787x; 1.0009x over previous
import jax
import jax.numpy as jnp
from jax import lax
from jax.experimental import pallas as pl
from jax.experimental.pallas import tpu as pltpu
from jax.experimental.pallas import tpu_sc as plsc

_B = 16
_C = 3
_H = 512
_W = 512

_P = 2
_NBUF = 6
_NCHUNK = (_B * _C) // _P


def _sc_lookup(cam_hbm, idi_hbm, dt_hbm,
               wc1_hbm, bc1_hbm, wi1_hbm, bi1_hbm,
               wc2_hbm, bc2_hbm, wi2_hbm, bi2_hbm,
               w_out, b_out,
               cam_v, idi_v, dt_v,
               g_cam, g_id, r_v, sem):
    wid = lax.axis_index("s") * 2 + lax.axis_index("c")

    def lookup(cam_tbl1, id_tbl1, cam_tbl2, id_tbl2, out_ref):
        cp = [pltpu.async_copy(src, dst, sem)
              for src, dst in ((cam_hbm, cam_v), (idi_hbm, idi_v),
                               (dt_hbm, dt_v))]
        for c in cp:
            c.wait()
        cam = cam_v[...]
        idi = idi_v[...]
        dt = dt_v[...]
        for i in range(_B):
            cam_i = cam[i]
            idi_i = idi[i]
            use1 = dt[i] == 0

            @pl.when(use1)
            def _():
                pltpu.async_copy(cam_tbl1.at[pl.ds(cam_i, 1)],
                                 g_cam.at[pl.ds(i, 1)], sem)
                pltpu.async_copy(id_tbl1.at[pl.ds(idi_i, 1)],
                                 g_id.at[pl.ds(i, 1)], sem)

            @pl.when(jnp.logical_not(use1))
            def _():
                pltpu.async_copy(cam_tbl2.at[pl.ds(cam_i, 1)],
                                 g_cam.at[pl.ds(i, 1)], sem)
                pltpu.async_copy(id_tbl2.at[pl.ds(idi_i, 1)],
                                 g_id.at[pl.ds(i, 1)], sem)
        for i in range(_B):
            pltpu.make_async_copy(cam_tbl1.at[pl.ds(0, 1)],
                                  g_cam.at[pl.ds(i, 1)], sem).wait()
            pltpu.make_async_copy(id_tbl1.at[pl.ds(0, 1)],
                                  g_id.at[pl.ds(i, 1)], sem).wait()

        samp = lax.iota(jnp.int32, _B)
        for c in range(_C):
            cvec = jnp.full((_B,), c, jnp.int32)
            v = (plsc.load_gather(g_cam, [samp, cvec])
                 + plsc.load_gather(g_id, [samp, cvec]))
            plsc.store_scatter(r_v, [samp * _C + c], v)
        pltpu.sync_copy(r_v, out_ref)

    @pl.when(wid == 0)
    def _():
        lookup(wc1_hbm, wi1_hbm, wc2_hbm, wi2_hbm, w_out)

    @pl.when(wid == 1)
    def _():
        lookup(bc1_hbm, bi1_hbm, bc2_hbm, bi2_hbm, b_out)


def _affine_body(w_ref, b_ref, img_ref, out_ref, buf_in, buf_out, sem_in, sem_out):
    def in_copy(k, s):
        return pltpu.make_async_copy(
            img_ref.at[pl.ds(k * _P, _P)], buf_in.at[s], sem_in.at[s])

    def out_copy(k, s):
        return pltpu.make_async_copy(
            buf_out.at[s], out_ref.at[pl.ds(k * _P, _P)], sem_out.at[s])

    for s in range(_NBUF):
        in_copy(s, s).start()
    for k in range(_NCHUNK):
        s = k % _NBUF
        in_copy(k, s).wait()
        if k >= _NBUF:
            out_copy(k - _NBUF, s).wait()
        w = w_ref[pl.ds(k * _P, _P)].reshape(_P, 1, 1)
        b = b_ref[pl.ds(k * _P, _P)].reshape(_P, 1, 1)
        buf_out[s] = buf_in[s] * w + b
        out_copy(k, s).start()
        if k + _NBUF < _NCHUNK:
            in_copy(k + _NBUF, s).start()
    for k in range(_NCHUNK - _NBUF, _NCHUNK):
        out_copy(k, k % _NBUF).wait()


def kernel(image, camindex, idindex, dataset_type,
           wcam1, bcam1, wident1, bident1,
           wcam2, bcam2, wident2, bident2):
    mesh = plsc.VectorSubcoreMesh(core_axis_name="c", subcore_axis_name="s")
    vec = jax.ShapeDtypeStruct((_B * _C,), jnp.float32)
    sc_fn = pl.kernel(
        _sc_lookup,
        out_type=[vec, vec],
        mesh=mesh,
        scratch_types=[
            pltpu.VMEM((_B,), jnp.int32),
            pltpu.VMEM((_B,), jnp.int32),
            pltpu.VMEM((_B,), jnp.int32),
            pltpu.VMEM((_B, _C), jnp.float32),
            pltpu.VMEM((_B, _C), jnp.float32),
            pltpu.VMEM((_B * _C,), jnp.float32),
            pltpu.SemaphoreType.DMA,
        ],
        name="colorcal_sc_lookup",
        compiler_params=pltpu.CompilerParams(needs_layout_passes=False,
                                             use_tc_tiling_on_sc=True),
    )
    w48, b48 = sc_fn(camindex, idindex, dataset_type,
                     wcam1, bcam1, wident1, bident1,
                     wcam2, bcam2, wident2, bident2)

    img = image.reshape(_B * _C, _H, _W)
    out = pl.pallas_call(
        _affine_body,
        in_specs=[
            pl.BlockSpec(memory_space=pltpu.VMEM),
            pl.BlockSpec(memory_space=pltpu.VMEM),
            pl.BlockSpec(memory_space=pl.ANY),
        ],
        out_specs=pl.BlockSpec(memory_space=pl.ANY),
        out_shape=jax.ShapeDtypeStruct((_B * _C, _H, _W), jnp.float32),
        scratch_shapes=[
            pltpu.VMEM((_NBUF, _P, _H, _W), jnp.float32),
            pltpu.VMEM((_NBUF, _P, _H, _W), jnp.float32),
            pltpu.SemaphoreType.DMA((_NBUF,)),
            pltpu.SemaphoreType.DMA((_NBUF,)),
        ],
        name="colorcal_affine",
    )(w48, b48, img)
    return out.reshape(_B, _C, _H, _W)


# speedup vs baseline: 1.6792x; 1.0003x over previous
import jax
import jax.numpy as jnp
from jax import lax
from jax.experimental import pallas as pl
from jax.experimental.pallas import tpu as pltpu
from jax.experimental.pallas import tpu_sc as plsc

_B = 16
_C = 3
_H = 512
_W = 512

_P = 4
_NBUF = 4
_NCHUNK = (_B * _C) // _P


def _sc_lookup(cam_hbm, idi_hbm, dt_hbm,
               wc1_hbm, bc1_hbm, wi1_hbm, bi1_hbm,
               wc2_hbm, bc2_hbm, wi2_hbm, bi2_hbm,
               w_out, b_out,
               cam_v, idi_v, dt_v,
               g_cam, g_id, r_v, sem):
    wid = lax.axis_index("s") * 2 + lax.axis_index("c")

    def lookup(cam_tbl1, id_tbl1, cam_tbl2, id_tbl2, out_ref):
        cp = [pltpu.async_copy(src, dst, sem)
              for src, dst in ((cam_hbm, cam_v), (idi_hbm, idi_v),
                               (dt_hbm, dt_v))]
        for c in cp:
            c.wait()
        cam = cam_v[...]
        idi = idi_v[...]
        dt = dt_v[...]
        for i in range(_B):
            cam_i = cam[i]
            idi_i = idi[i]
            use1 = dt[i] == 0

            @pl.when(use1)
            def _():
                pltpu.async_copy(cam_tbl1.at[pl.ds(cam_i, 1)],
                                 g_cam.at[pl.ds(i, 1)], sem)
                pltpu.async_copy(id_tbl1.at[pl.ds(idi_i, 1)],
                                 g_id.at[pl.ds(i, 1)], sem)

            @pl.when(jnp.logical_not(use1))
            def _():
                pltpu.async_copy(cam_tbl2.at[pl.ds(cam_i, 1)],
                                 g_cam.at[pl.ds(i, 1)], sem)
                pltpu.async_copy(id_tbl2.at[pl.ds(idi_i, 1)],
                                 g_id.at[pl.ds(i, 1)], sem)
        for i in range(_B):
            pltpu.make_async_copy(cam_tbl1.at[pl.ds(0, 1)],
                                  g_cam.at[pl.ds(i, 1)], sem).wait()
            pltpu.make_async_copy(id_tbl1.at[pl.ds(0, 1)],
                                  g_id.at[pl.ds(i, 1)], sem).wait()

        samp = lax.iota(jnp.int32, _B)
        for c in range(_C):
            cvec = jnp.full((_B,), c, jnp.int32)
            v = (plsc.load_gather(g_cam, [samp, cvec])
                 + plsc.load_gather(g_id, [samp, cvec]))
            plsc.store_scatter(r_v, [samp * _C + c], v)
        pltpu.sync_copy(r_v, out_ref)

    @pl.when(wid == 0)
    def _():
        lookup(wc1_hbm, wi1_hbm, wc2_hbm, wi2_hbm, w_out)

    @pl.when(wid == 1)
    def _():
        lookup(bc1_hbm, bi1_hbm, bc2_hbm, bi2_hbm, b_out)


def _affine_body(w_ref, b_ref, img_ref, out_ref, buf_in, buf_out, sem_in, sem_out):
    def in_copy(k, s):
        return pltpu.make_async_copy(
            img_ref.at[pl.ds(k * _P, _P)], buf_in.at[s], sem_in.at[s])

    def out_copy(k, s):
        return pltpu.make_async_copy(
            buf_out.at[s], out_ref.at[pl.ds(k * _P, _P)], sem_out.at[s])

    for s in range(_NBUF):
        in_copy(s, s).start()
    for k in range(_NCHUNK):
        s = k % _NBUF
        in_copy(k, s).wait()
        if k >= _NBUF:
            out_copy(k - _NBUF, s).wait()
        w = w_ref[pl.ds(k * _P, _P)].reshape(_P, 1, 1)
        b = b_ref[pl.ds(k * _P, _P)].reshape(_P, 1, 1)
        buf_out[s] = buf_in[s] * w + b
        out_copy(k, s).start()
        if k + _NBUF < _NCHUNK:
            in_copy(k + _NBUF, s).start()
    for k in range(_NCHUNK - _NBUF, _NCHUNK):
        out_copy(k, k % _NBUF).wait()


def kernel(image, camindex, idindex, dataset_type,
           wcam1, bcam1, wident1, bident1,
           wcam2, bcam2, wident2, bident2):
    mesh = plsc.VectorSubcoreMesh(core_axis_name="c", subcore_axis_name="s")
    vec = jax.ShapeDtypeStruct((_B * _C,), jnp.float32)
    sc_fn = pl.kernel(
        _sc_lookup,
        out_type=[vec, vec],
        mesh=mesh,
        scratch_types=[
            pltpu.VMEM((_B,), jnp.int32),
            pltpu.VMEM((_B,), jnp.int32),
            pltpu.VMEM((_B,), jnp.int32),
            pltpu.VMEM((_B, _C), jnp.float32),
            pltpu.VMEM((_B, _C), jnp.float32),
            pltpu.VMEM((_B * _C,), jnp.float32),
            pltpu.SemaphoreType.DMA,
        ],
        name="colorcal_sc_lookup",
        compiler_params=pltpu.CompilerParams(needs_layout_passes=False,
                                             use_tc_tiling_on_sc=True),
    )
    w48, b48 = sc_fn(camindex, idindex, dataset_type,
                     wcam1, bcam1, wident1, bident1,
                     wcam2, bcam2, wident2, bident2)

    img = image.reshape(_B * _C, _H, _W)
    out = pl.pallas_call(
        _affine_body,
        in_specs=[
            pl.BlockSpec(memory_space=pltpu.VMEM),
            pl.BlockSpec(memory_space=pltpu.VMEM),
            pl.BlockSpec(memory_space=pl.ANY),
        ],
        out_specs=pl.BlockSpec(memory_space=pl.ANY),
        out_shape=jax.ShapeDtypeStruct((_B * _C, _H, _W), jnp.float32),
        scratch_shapes=[
            pltpu.VMEM((_NBUF, _P, _H, _W), jnp.float32),
            pltpu.VMEM((_NBUF, _P, _H, _W), jnp.float32),
            pltpu.SemaphoreType.DMA((_NBUF,)),
            pltpu.SemaphoreType.DMA((_NBUF,)),
        ],
        name="colorcal_affine",
    )(w48, b48, img)
    return out.reshape(_B, _C, _H, _W)
